# baseline (speedup 1.0000x reference)
import functools

import jax
import jax.numpy as jnp
from jax.experimental import pallas as pl
from jax.experimental.pallas import tpu as pltpu


def _leaky(x):
    return jnp.where(x >= 0, x, 0.01 * x)


def _ln(x, w, b):
    m = jnp.mean(x, axis=-1, keepdims=True)
    v = jnp.var(x, axis=-1, keepdims=True)
    return (x - m) / jnp.sqrt(v + 1e-5) * w + b


def _emb(x, p):
    x = _ln(x, p['ln_w'], p['ln_b'])
    x = _leaky(x @ p['W1'].T + p['b1'])
    return _leaky(x @ p['W2'].T + p['b2'])


def _bconv(left, right, src, dst, p):
    l = left @ p['fl_W'].T + p['fl_b']
    r = right @ p['fr_W'].T
    pre = _leaky(_ln(l[src] + r[dst], p['ln1_w'], p['ln1_b']))
    n = right.shape[0]
    ssum = jax.ops.segment_sum(pre, dst, num_segments=n)
    cnt = jax.ops.segment_sum(jnp.ones((dst.shape[0],), pre.dtype), dst, num_segments=n)
    cnt_ = jnp.maximum(cnt, 1.0)[:, None]
    agg = (ssum / cnt_) @ p['ff_W'].T + jnp.where(cnt[:, None] > 0, p['ff_b'], 0.0)
    post = _ln(agg, p['post_w'], p['post_b'])
    h = jnp.concatenate([post, right], axis=-1)
    return (_leaky(h @ p['o1_W'].T + p['o1_b'])) @ p['o2_W'].T + p['o2_b']


def _head_body(x_ref, w0_ref, b0_ref, wf_ref, bf_ref, o_ref):
    x = x_ref[...]
    h = x @ w0_ref[...].T + b0_ref[...]
    h = jnp.where(h >= 0, h, 0.01 * h)
    o_ref[...] = h @ wf_ref[...] + bf_ref[0, 0]


def _head_full(cut, h):
    n, e = cut.shape
    blk = 2000
    return pl.pallas_call(
        _head_body,
        grid=(n // blk,),
        in_specs=[
            pl.BlockSpec((blk, e), lambda i: (i, 0)),
            pl.BlockSpec((e, e), lambda i: (0, 0)),
            pl.BlockSpec((1, e), lambda i: (0, 0)),
            pl.BlockSpec((e, 1), lambda i: (0, 0)),
            pl.BlockSpec((1, 1), lambda i: (0, 0)),
        ],
        out_specs=pl.BlockSpec((blk, 1), lambda i: (i, 0)),
        out_shape=jax.ShapeDtypeStruct((n, 1), jnp.float32),
    )(cut, h['W0'], h['b0'].reshape(1, e), h['Wf'].T, h['bf'].reshape(1, 1))


def kernel(row_features, variable_features, cut_features, edge_index,
           cut_row_edge_index, cut_col_edge_index, edge_features,
           cut_row_edge_features, cut_col_edge_features, params):
    p = params
    cons = _emb(row_features, p['emb_cons'])
    var = _emb(variable_features, p['emb_var'])
    cut = _emb(cut_features, p['emb_cut'])
    ei0, ei1 = edge_index[0], edge_index[1]
    cr0, cr1 = cut_row_edge_index[0], cut_row_edge_index[1]
    cc0, cc1 = cut_col_edge_index[0], cut_col_edge_index[1]
    cons = _bconv(var, cons, ei1, ei0, p['conv_v_to_c'])
    cons = _bconv(cut, cons, cr0, cr1, p['conv_cut_to_c'])
    var = _bconv(cons, var, ei0, ei1, p['conv_c_to_v'])
    var = _bconv(cut, var, cc0, cc1, p['conv_cut_to_v'])
    cut = _bconv(var, cut, cc1, cc0, p['conv_v_to_cut'])
    cut = _bconv(cons, cut, cr1, cr0, p['conv_c_to_cut'])
    return _head_full(cut, p['head'])[:, 0]


# speedup vs baseline: 1.2582x; 1.2582x over previous
import functools

import jax
import jax.numpy as jnp
from jax import lax
from jax.experimental import pallas as pl
from jax.experimental.pallas import tpu as pltpu
from jax.experimental.pallas import tpu_sc as plsc

_NS = 16
_WIN = 128


def _leaky(x):
    return jnp.where(x >= 0, x, 0.01 * x)


def _ln(x, w, b):
    m = jnp.mean(x, axis=-1, keepdims=True)
    v = jnp.var(x, axis=-1, keepdims=True)
    return (x - m) / jnp.sqrt(v + 1e-5) * w + b


def _emb(x, p):
    x = _ln(x, p['ln_w'], p['ln_b'])
    x = _leaky(x @ p['W1'].T + p['b1'])
    return _leaky(x @ p['W2'].T + p['b2'])


@functools.lru_cache(maxsize=None)
def _edge_kernel(nl, nr, e):
    half = nr // 2
    halfp = ((half + 1 + 127) // 128) * 128
    nwin = e // _WIN
    per = halfp // _NS
    mesh = plsc.VectorSubcoreMesh(core_axis_name="c", subcore_axis_name="s")

    def body(l_hbm, r_hbm, src_hbm, dst_hbm, z64_hbm, z8_hbm, ones_hbm,
             lnw_hbm, lnb_hbm, sum_out, cnt_out,
             table, cntt, sidx, didx, lidx, rows_l, rows_r, ones_v,
             lnw_v, lnb_v, sem0, sem1):
        c = lax.axis_index("c")
        s = lax.axis_index("s")
        base = c * half
        lo = s * per
        pltpu.sync_copy(z64_hbm, table.at[pl.ds(lo, per)])
        pltpu.sync_copy(z8_hbm, cntt.at[pl.ds(lo, per)])
        pltpu.sync_copy(ones_hbm, ones_v)
        pltpu.sync_copy(lnw_hbm, lnw_v)
        pltpu.sync_copy(lnb_hbm, lnb_v)
        plsc.subcore_barrier()

        nb, rem = nwin // _NS, nwin % _NS
        lo_w = s * nb + jnp.minimum(s, rem)
        n_w = nb + jnp.where(s < rem, 1, 0)
        iota = lax.iota(jnp.int32, 16)
        dummy = half + (iota & 7)
        shuf = [iota ^ k for k in (8, 4, 2, 1)]

        dnums = lax.GatherDimensionNumbers(
            offset_dims=(), collapsed_slice_dims=(0,), start_index_map=(0,))

        def _rsqrt(a):
            a = a * jnp.float32(2.0 ** 30)
            u = jnp.full((16,), jnp.float32(2.0 ** 15))
            for k in (16, 8, 4, 2, 1):
                c = a >= jnp.float32(4.0 ** k)
                a = jnp.where(c, a * jnp.float32(4.0 ** -k), a)
                u = jnp.where(c, u * jnp.float32(2.0 ** -k), u)
            t = jnp.float32(1.30865787) + a * (
                jnp.float32(-0.39507222) + a * jnp.float32(0.04938426))
            t = t * (1.5 - (0.5 * a) * t * t)
            t = t * (1.5 - (0.5 * a) * t * t)
            return t * u

        def _allsum(x):
            for ix in shuf:
                x = x + lax.gather(
                    x, ix[:, None], dnums, slice_sizes=(1,),
                    mode=lax.GatherScatterMode.PROMISE_IN_BOUNDS)
            return x

        def win(widx, carry):
            pltpu.sync_copy(src_hbm.at[widx], sidx)
            pltpu.sync_copy(dst_hbm.at[widx], didx)
            ca = pltpu.async_copy(l_hbm.at[sidx], rows_l, sem0)
            cb = pltpu.async_copy(r_hbm.at[didx], rows_r, sem1)
            ca.wait()
            cb.wait()
            for i in range(_WIN // 16):
                d16 = didx[pl.ds(i * 16, 16)]
                loc = d16 - base
                inr = (loc >= 0) & (loc < half)
                lidx[pl.ds(i * 16, 16)] = jnp.where(inr, loc, dummy)

            def edge(ei, carry2):
                x0 = rows_l[ei, pl.ds(0, 16)] + rows_r[ei, pl.ds(0, 16)]
                x1 = rows_l[ei, pl.ds(16, 16)] + rows_r[ei, pl.ds(16, 16)]
                x2 = rows_l[ei, pl.ds(32, 16)] + rows_r[ei, pl.ds(32, 16)]
                x3 = rows_l[ei, pl.ds(48, 16)] + rows_r[ei, pl.ds(48, 16)]
                tot = _allsum(x0 + x1 + x2 + x3)
                m = tot * (1.0 / 64.0)
                y0, y1, y2, y3 = x0 - m, x1 - m, x2 - m, x3 - m
                tot2 = _allsum(y0 * y0 + y1 * y1 + y2 * y2 + y3 * y3)
                av = tot2 * (1.0 / 64.0) + 1e-5
                f = _rsqrt(av)
                for j, yj in enumerate((y0, y1, y2, y3)):
                    z = yj * f * lnw_v[j] + lnb_v[j]
                    rows_l[ei, pl.ds(j * 16, 16)] = (
                        jnp.maximum(z, 0.0) + 0.01 * jnp.minimum(z, 0.0))
                return carry2

            lax.fori_loop(0, _WIN, edge, 0)
            pltpu.sync_copy(rows_l, table.at[lidx], add=True)
            pltpu.sync_copy(ones_v, cntt.at[lidx], add=True)
            return carry

        lax.fori_loop(lo_w, lo_w + n_w, win, 0)
        plsc.subcore_barrier()
        pltpu.sync_copy(table.at[pl.ds(lo, per)],
                        sum_out.at[pl.ds(c * halfp + lo, per)])
        pltpu.sync_copy(cntt.at[pl.ds(lo, per)],
                        cnt_out.at[pl.ds(c * halfp + lo, per)])

    return pl.kernel(
        body,
        mesh=mesh,
        compiler_params=pltpu.CompilerParams(use_tc_tiling_on_sc=False),
        out_type=[jax.ShapeDtypeStruct((2 * halfp, 64), jnp.float32),
                  jax.ShapeDtypeStruct((2 * halfp, 8), jnp.float32)],
        scratch_types=[
            pltpu.VMEM_SHARED((halfp, 64), jnp.float32),
            pltpu.VMEM_SHARED((halfp, 8), jnp.float32),
            pltpu.VMEM((_WIN,), jnp.int32),
            pltpu.VMEM((_WIN,), jnp.int32),
            pltpu.VMEM((_WIN,), jnp.int32),
            pltpu.VMEM((_WIN, 64), jnp.float32),
            pltpu.VMEM((_WIN, 64), jnp.float32),
            pltpu.VMEM((_WIN, 8), jnp.float32),
            pltpu.VMEM((4, 16), jnp.float32),
            pltpu.VMEM((4, 16), jnp.float32),
            pltpu.SemaphoreType.DMA,
            pltpu.SemaphoreType.DMA,
        ],
    )


def _edge_stage(l, r, src, dst, lnw, lnb):
    nl, nr, e = l.shape[0], r.shape[0], src.shape[0]
    half = nr // 2
    halfp = ((half + 1 + 127) // 128) * 128
    per = halfp // _NS
    k = _edge_kernel(nl, nr, e)
    src2 = src.astype(jnp.int32).reshape(e // _WIN, _WIN)
    dst2 = dst.astype(jnp.int32).reshape(e // _WIN, _WIN)
    ssum, cntt = k(l, r, src2, dst2,
                   jnp.zeros((per, 64), jnp.float32),
                   jnp.zeros((per, 8), jnp.float32),
                   jnp.ones((_WIN, 8), jnp.float32),
                   lnw.reshape(4, 16), lnb.reshape(4, 16))
    ssum = jnp.concatenate([ssum[:half], ssum[halfp:halfp + half]], axis=0)
    cnt = jnp.concatenate([cntt[:half, 0], cntt[halfp:halfp + half, 0]])
    return ssum, cnt


def _bconv(left, right, src, dst, p):
    l = left @ p['fl_W'].T + p['fl_b']
    r = right @ p['fr_W'].T
    ssum, cnt = _edge_stage(l, r, src, dst, p['ln1_w'], p['ln1_b'])
    cnt_ = jnp.maximum(cnt, 1.0)[:, None]
    agg = (ssum / cnt_) @ p['ff_W'].T + jnp.where(cnt[:, None] > 0, p['ff_b'], 0.0)
    post = _ln(agg, p['post_w'], p['post_b'])
    h = jnp.concatenate([post, right], axis=-1)
    return (_leaky(h @ p['o1_W'].T + p['o1_b'])) @ p['o2_W'].T + p['o2_b']


def _head_body(x_ref, w0_ref, b0_ref, wf_ref, bf_ref, o_ref):
    x = x_ref[...]
    h = x @ w0_ref[...].T + b0_ref[...]
    h = jnp.where(h >= 0, h, 0.01 * h)
    o_ref[...] = h @ wf_ref[...] + bf_ref[0, 0]


def _head_full(cut, h):
    n, e = cut.shape
    blk = 2000
    return pl.pallas_call(
        _head_body,
        grid=(n // blk,),
        in_specs=[
            pl.BlockSpec((blk, e), lambda i: (i, 0)),
            pl.BlockSpec((e, e), lambda i: (0, 0)),
            pl.BlockSpec((1, e), lambda i: (0, 0)),
            pl.BlockSpec((e, 1), lambda i: (0, 0)),
            pl.BlockSpec((1, 1), lambda i: (0, 0)),
        ],
        out_specs=pl.BlockSpec((blk, 1), lambda i: (i, 0)),
        out_shape=jax.ShapeDtypeStruct((n, 1), jnp.float32),
    )(cut, h['W0'], h['b0'].reshape(1, e), h['Wf'].T, h['bf'].reshape(1, 1))


def kernel(row_features, variable_features, cut_features, edge_index,
           cut_row_edge_index, cut_col_edge_index, edge_features,
           cut_row_edge_features, cut_col_edge_features, params):
    p = params
    cons = _emb(row_features, p['emb_cons'])
    var = _emb(variable_features, p['emb_var'])
    cut = _emb(cut_features, p['emb_cut'])
    ei0, ei1 = edge_index[0], edge_index[1]
    cr0, cr1 = cut_row_edge_index[0], cut_row_edge_index[1]
    cc0, cc1 = cut_col_edge_index[0], cut_col_edge_index[1]
    cons = _bconv(var, cons, ei1, ei0, p['conv_v_to_c'])
    cons = _bconv(cut, cons, cr0, cr1, p['conv_cut_to_c'])
    var = _bconv(cons, var, ei0, ei1, p['conv_c_to_v'])
    var = _bconv(cut, var, cc0, cc1, p['conv_cut_to_v'])
    cut = _bconv(var, cut, cc1, cc0, p['conv_v_to_cut'])
    cut = _bconv(cons, cut, cr1, cr0, p['conv_c_to_cut'])
    return _head_full(cut, p['head'])[:, 0]
